# baseline (speedup 1.0000x reference)
import jax
import jax.numpy as jnp
from jax import lax
from jax.experimental import pallas as pl
from jax.experimental.pallas import tpu as pltpu
from jax.experimental.pallas import tpu_sc as plsc

BATCH = 4096
EMBED_DIM = 128
NUM_CORES = 2
NUM_SUBCORES = 16
NUM_WORKERS = NUM_CORES * NUM_SUBCORES
TPW = BATCH // NUM_WORKERS
GROUPS = TPW // 16

BITREV = (0, 8, 4, 12, 2, 10, 6, 14, 1, 9, 5, 13, 3, 11, 7, 15)


def _sc_score_kernel(heads_hbm, rels_hbm, tails_hbm, entity_hbm, relation_hbm,
                     out_hbm,
                     hidx_v, ridx_v, tidx_v, hrows_v, rrows_v, trows_v,
                     scores_v, rot_v, sem, sem_b, sem_i):
    wid = lax.axis_index("s") * NUM_CORES + lax.axis_index("c")
    iota16 = lax.iota(jnp.int32, 16)

    base = pl.multiple_of(wid * TPW, 8)
    ci_h = pltpu.async_copy(heads_hbm.at[pl.ds(base, TPW)], hidx_v, sem_i)
    ci_r = pltpu.async_copy(rels_hbm.at[pl.ds(base, TPW)], ridx_v, sem_i)
    ci_t = pltpu.async_copy(tails_hbm.at[pl.ds(base, TPW)], tidx_v, sem_i)
    half = TPW // 2
    ci_h.wait()
    cp_ha = pltpu.async_copy(entity_hbm.at[hidx_v.at[pl.ds(0, half)]],
                             hrows_v.at[pl.ds(0, half)], sem)
    ci_r.wait()
    cp_ra = pltpu.async_copy(relation_hbm.at[ridx_v.at[pl.ds(0, half)]],
                             rrows_v.at[pl.ds(0, half)], sem)
    ci_t.wait()
    cp_ta = pltpu.async_copy(entity_hbm.at[tidx_v.at[pl.ds(0, half)]],
                             trows_v.at[pl.ds(0, half)], sem)
    cp_hb = pltpu.async_copy(entity_hbm.at[hidx_v.at[pl.ds(half, half)]],
                             hrows_v.at[pl.ds(half, half)], sem_b)
    cp_rb = pltpu.async_copy(relation_hbm.at[ridx_v.at[pl.ds(half, half)]],
                             rrows_v.at[pl.ds(half, half)], sem_b)
    cp_tb = pltpu.async_copy(entity_hbm.at[tidx_v.at[pl.ds(half, half)]],
                             trows_v.at[pl.ds(half, half)], sem_b)

    m1 = iota16 < 8
    m2 = (iota16 & 4) == 0
    m3 = (iota16 & 2) == 0
    m4 = (iota16 & 1) == 0
    nslots = [0]

    def fold(v, shift):
        slot = nslots[0]
        nslots[0] = (slot + 1) % 32
        rot_v[slot, pl.ds(0, 16)] = v
        rot_v[slot, pl.ds(16, 16)] = v
        return v + rot_v[slot, pl.ds(shift, 16)]

    def score_group(g, carry):
        def leaf(l):
            i = g * 16 + BITREV[l]
            acc = None
            for c in range(EMBED_DIM // 16):
                h = hrows_v[i, pl.ds(c * 16, 16)]
                r = rrows_v[i, pl.ds(c * 16, 16)]
                t = trows_v[i, pl.ds(c * 16, 16)]
                d = h + r - t
                acc = d * d if acc is None else acc + d * d
            return acc

        a = [jnp.where(m1, fold(leaf(2 * p), 8), fold(leaf(2 * p + 1), 8))
             for p in range(8)]
        b = [jnp.where(m2, fold(a[2 * p], 4), fold(a[2 * p + 1], 12))
             for p in range(4)]
        c = [jnp.where(m3, fold(b[2 * p], 2), fold(b[2 * p + 1], 14))
             for p in range(2)]
        x = jnp.where(m4, fold(c[0], 1), fold(c[1], 15))

        x = x + 1e-12
        bits = lax.bitcast_convert_type(x, jnp.int32)
        bits = 0x5F3759DF - lax.shift_right_logical(bits, 1)
        y = lax.bitcast_convert_type(bits, jnp.float32)
        for _ in range(3):
            y = y * (1.5 - 0.5 * x * y * y)
        scores_v[pl.ds(g * 16, 16)] = -(x * y)
        return carry

    cp_ha.wait()
    cp_ra.wait()
    cp_ta.wait()
    lax.fori_loop(0, GROUPS // 2, score_group, 0)
    cp_hb.wait()
    cp_rb.wait()
    cp_tb.wait()
    lax.fori_loop(GROUPS // 2, GROUPS, score_group, 0)

    pltpu.sync_copy(scores_v, out_hbm.at[pl.ds(base, TPW)])


@jax.jit
def _sc_score(heads, rels, tails, entity_emb, relation_emb):
    mesh = plsc.VectorSubcoreMesh(core_axis_name="c", subcore_axis_name="s")
    return pl.kernel(
        _sc_score_kernel,
        out_type=jax.ShapeDtypeStruct((BATCH,), jnp.float32),
        mesh=mesh,
        scratch_types=[
            pltpu.VMEM((TPW,), jnp.int32),
            pltpu.VMEM((TPW,), jnp.int32),
            pltpu.VMEM((TPW,), jnp.int32),
            pltpu.VMEM((TPW, EMBED_DIM), jnp.float32),
            pltpu.VMEM((TPW, EMBED_DIM), jnp.float32),
            pltpu.VMEM((TPW, EMBED_DIM), jnp.float32),
            pltpu.VMEM((TPW,), jnp.float32),
            pltpu.VMEM((32, 32), jnp.float32),
            pltpu.SemaphoreType.DMA,
            pltpu.SemaphoreType.DMA,
            pltpu.SemaphoreType.DMA,
        ],
    )(heads, rels, tails, entity_emb, relation_emb)


def kernel(triples, entity_emb, relation_emb):
    trip = triples.astype(jnp.int32)
    return _sc_score(trip[:, 0], trip[:, 1], trip[:, 2],
                     entity_emb, relation_emb)


# speedup vs baseline: 1.0444x; 1.0444x over previous
import jax
import jax.numpy as jnp
from jax import lax
from jax.experimental import pallas as pl
from jax.experimental.pallas import tpu as pltpu
from jax.experimental.pallas import tpu_sc as plsc

BATCH = 4096
EMBED_DIM = 128
NUM_CORES = 2
NUM_SUBCORES = 16
NUM_WORKERS = NUM_CORES * NUM_SUBCORES
TPW = BATCH // NUM_WORKERS
GROUPS = TPW // 16

BITREV = (0, 8, 4, 12, 2, 10, 6, 14, 1, 9, 5, 13, 3, 11, 7, 15)


def _sc_score_kernel(heads_hbm, rels_hbm, tails_hbm, entity_hbm, relation_hbm,
                     out_hbm,
                     hidx_v, ridx_v, tidx_v, hrows_v, rrows_v, trows_v,
                     scores_v, rot_v, sem, sem_i):
    wid = lax.axis_index("s") * NUM_CORES + lax.axis_index("c")
    iota16 = lax.iota(jnp.int32, 16)

    base = pl.multiple_of(wid * TPW, 8)
    ci_h = pltpu.async_copy(heads_hbm.at[pl.ds(base, TPW)], hidx_v, sem_i)
    ci_r = pltpu.async_copy(rels_hbm.at[pl.ds(base, TPW)], ridx_v, sem_i)
    ci_t = pltpu.async_copy(tails_hbm.at[pl.ds(base, TPW)], tidx_v, sem_i)
    ci_h.wait()
    cp_h = pltpu.async_copy(entity_hbm.at[hidx_v], hrows_v, sem)
    ci_r.wait()
    cp_r = pltpu.async_copy(relation_hbm.at[ridx_v], rrows_v, sem)
    ci_t.wait()
    cp_t = pltpu.async_copy(entity_hbm.at[tidx_v], trows_v, sem)

    m1 = iota16 < 8
    m2 = (iota16 & 4) == 0
    m3 = (iota16 & 2) == 0
    m4 = (iota16 & 1) == 0
    nslots = [0]

    def fold(v, shift):
        slot = nslots[0]
        nslots[0] = (slot + 1) % 32
        rot_v[slot, pl.ds(0, 16)] = v
        rot_v[slot, pl.ds(16, 16)] = v
        return v + rot_v[slot, pl.ds(shift, 16)]

    def score_group(g, carry):
        def leaf(l):
            i = g * 16 + BITREV[l]
            acc = None
            for c in range(EMBED_DIM // 16):
                h = hrows_v[i, pl.ds(c * 16, 16)]
                r = rrows_v[i, pl.ds(c * 16, 16)]
                t = trows_v[i, pl.ds(c * 16, 16)]
                d = h + r - t
                acc = d * d if acc is None else acc + d * d
            return acc

        a = [jnp.where(m1, fold(leaf(2 * p), 8), fold(leaf(2 * p + 1), 8))
             for p in range(8)]
        b = [jnp.where(m2, fold(a[2 * p], 4), fold(a[2 * p + 1], 12))
             for p in range(4)]
        c = [jnp.where(m3, fold(b[2 * p], 2), fold(b[2 * p + 1], 14))
             for p in range(2)]
        x = jnp.where(m4, fold(c[0], 1), fold(c[1], 15))

        x = x + 1e-12
        bits = lax.bitcast_convert_type(x, jnp.int32)
        bits = 0x5F3759DF - lax.shift_right_logical(bits, 1)
        y = lax.bitcast_convert_type(bits, jnp.float32)
        for _ in range(3):
            y = y * (1.5 - 0.5 * x * y * y)
        scores_v[pl.ds(g * 16, 16)] = -(x * y)
        return carry

    cp_h.wait()
    cp_r.wait()
    cp_t.wait()
    lax.fori_loop(0, GROUPS, score_group, 0)

    pltpu.sync_copy(scores_v, out_hbm.at[pl.ds(base, TPW)])


@jax.jit
def _sc_score(heads, rels, tails, entity_emb, relation_emb):
    mesh = plsc.VectorSubcoreMesh(core_axis_name="c", subcore_axis_name="s")
    return pl.kernel(
        _sc_score_kernel,
        out_type=jax.ShapeDtypeStruct((BATCH,), jnp.float32),
        mesh=mesh,
        scratch_types=[
            pltpu.VMEM((TPW,), jnp.int32),
            pltpu.VMEM((TPW,), jnp.int32),
            pltpu.VMEM((TPW,), jnp.int32),
            pltpu.VMEM((TPW, EMBED_DIM), jnp.float32),
            pltpu.VMEM((TPW, EMBED_DIM), jnp.float32),
            pltpu.VMEM((TPW, EMBED_DIM), jnp.float32),
            pltpu.VMEM((TPW,), jnp.float32),
            pltpu.VMEM((32, 32), jnp.float32),
            pltpu.SemaphoreType.DMA,
            pltpu.SemaphoreType.DMA,
        ],
    )(heads, rels, tails, entity_emb, relation_emb)


def kernel(triples, entity_emb, relation_emb):
    trip = triples.astype(jnp.int32)
    return _sc_score(trip[:, 0], trip[:, 1], trip[:, 2],
                     entity_emb, relation_emb)
